# baseline (speedup 1.0000x reference)
import functools

import jax
import jax.numpy as jnp
from jax import lax
from jax.experimental import pallas as pl
from jax.experimental.pallas import tpu as pltpu
from jax.experimental.pallas import tpu_sc as plsc

_D = 16
_CH = 8
_SLICES = ((0, 104), (104, 96))


def _build_sc(B, L, nc, ns):
    nw = nc * ns
    rw = B // nw
    nchunk = rw // _CH
    grp = rw // 128

    mesh = plsc.VectorSubcoreMesh(core_axis_name="c", subcore_axis_name="s")

    @functools.partial(
        pl.kernel,
        out_type=(
            jax.ShapeDtypeStruct((B, _D), jnp.float32),
            jax.ShapeDtypeStruct((B, _D), jnp.float32),
            jax.ShapeDtypeStruct((B, _D), jnp.float32),
        ),
        mesh=mesh,
        compiler_params=pltpu.CompilerParams(use_tc_tiling_on_sc=False),
        scratch_types=(
            pltpu.VMEM((2, _CH, 256), jnp.int32),
            pltpu.VMEM((2, _CH, L, _D), jnp.float32),
            pltpu.VMEM((_CH, _D), jnp.float32),
            pltpu.VMEM((grp, 128), jnp.int32),
            pltpu.VMEM((rw, _D), jnp.float32),
            pltpu.VMEM((grp, 128), jnp.int32),
            pltpu.VMEM((rw, _D), jnp.float32),
            pltpu.SemaphoreType.DMA,
            pltpu.SemaphoreType.DMA,
        ),
    )
    def sc_fn(uid_h, hist_h, tgt_h, ut_h, it_h, uo_h, ho_h, to_h,
              hidx, hrows, hsum, uidx, urows, tidx, trows, hsem, gsem):
        wid = lax.axis_index("s") * nc + lax.axis_index("c")
        base = wid * rw

        pltpu.sync_copy(uid_h.at[pl.ds(wid * grp, grp)], uidx)
        pltpu.sync_copy(tgt_h.at[pl.ds(wid * grp, grp)], tidx)
        handles = []
        for j in range(grp):
            handles.append(pltpu.async_copy(
                ut_h.at[uidx.at[j]], urows.at[pl.ds(j * 128, 128)], gsem))
            handles.append(pltpu.async_copy(
                it_h.at[tidx.at[j]], trows.at[pl.ds(j * 128, 128)], gsem))
        for h in handles:
            h.wait()
        pltpu.sync_copy(urows, uo_h.at[pl.ds(base, rw)])
        pltpu.sync_copy(trows, to_h.at[pl.ds(base, rw)])

        zero = jnp.zeros((_D,), jnp.float32)

        def do_chunk(bb, chunk):
            row0 = base + chunk * _CH
            pltpu.sync_copy(hist_h.at[pl.ds(row0, _CH)], hidx.at[bb])
            hh = []
            for i in range(_CH):
                for (o, n) in _SLICES:
                    hh.append(pltpu.async_copy(
                        it_h.at[hidx.at[bb, i, pl.ds(o, n)]],
                        hrows.at[bb, i, pl.ds(o, n)],
                        hsem))
            for h in hh:
                h.wait()
            for i in range(_CH):
                def acc(j, a, i=i):
                    a0, a1, a2, a3 = a
                    b0 = j * 20
                    for k in range(0, 20, 4):
                        a0 = a0 + hrows[bb, i, b0 + k]
                        a1 = a1 + hrows[bb, i, b0 + k + 1]
                        a2 = a2 + hrows[bb, i, b0 + k + 2]
                        a3 = a3 + hrows[bb, i, b0 + k + 3]
                    return a0, a1, a2, a3
                a0, a1, a2, a3 = lax.fori_loop(
                    0, L // 20, acc, (zero, zero, zero, zero))
                hsum[i] = (a0 + a1) + (a2 + a3)
            pltpu.sync_copy(hsum, ho_h.at[pl.ds(row0, _CH)])

        def body(g, c):
            for bb in range(2):
                do_chunk(bb, g * 2 + bb)
            return c

        lax.fori_loop(0, nchunk // 2, body, 0)

    return sc_fn


def _tc_mlp(urows, hsums, lenf, trows, W1, b1, W2, b2):
    B = urows.shape[0]
    h1 = W1.shape[1]
    grid = 8
    rb = B // grid

    def body(u_ref, h_ref, l_ref, t_ref, w1_ref, b1_ref, w2_ref, b2_ref,
             uv_ref, iv_ref):
        denom = jnp.maximum(l_ref[...], 1.0)
        x = jnp.concatenate([u_ref[...], h_ref[...] / denom], axis=1)
        h = jnp.dot(x, w1_ref[...], preferred_element_type=jnp.float32)
        h = jnp.maximum(h + b1_ref[...], 0.0)
        h = jnp.dot(h, w2_ref[...], preferred_element_type=jnp.float32)
        h = jnp.maximum(h + b2_ref[...], 0.0)
        n = jnp.sqrt(jnp.sum(h * h, axis=1, keepdims=True))
        uv_ref[...] = h / jnp.maximum(n, 1e-12)
        t = t_ref[...]
        tn = jnp.sqrt(jnp.sum(t * t, axis=1, keepdims=True))
        iv_ref[...] = t / jnp.maximum(tn, 1e-12)

    out = pl.pallas_call(
        body,
        grid=(grid,),
        in_specs=[
            pl.BlockSpec((rb, _D), lambda i: (i, 0)),
            pl.BlockSpec((rb, _D), lambda i: (i, 0)),
            pl.BlockSpec((rb, 1), lambda i: (i, 0)),
            pl.BlockSpec((rb, _D), lambda i: (i, 0)),
            pl.BlockSpec((2 * _D, h1), lambda i: (0, 0)),
            pl.BlockSpec((1, h1), lambda i: (0, 0)),
            pl.BlockSpec((h1, _D), lambda i: (0, 0)),
            pl.BlockSpec((1, _D), lambda i: (0, 0)),
        ],
        out_specs=[
            pl.BlockSpec((rb, _D), lambda i: (i, 0)),
            pl.BlockSpec((rb, _D), lambda i: (i, 0)),
        ],
        out_shape=[
            jax.ShapeDtypeStruct((B, _D), jnp.float32),
            jax.ShapeDtypeStruct((B, _D), jnp.float32),
        ],
    )(urows, hsums, lenf, trows, W1, b1, W2, b2)
    return tuple(out)


def kernel(user_id, hist_item, hist_len, target_item, user_table, item_table,
           W1, b1, W2, b2):
    B, L = hist_item.shape
    info = plsc.get_sparse_core_info()
    nc, ns = info.num_cores, info.num_subcores
    uid2 = user_id.astype(jnp.int32).reshape(B // 128, 128)
    tgt2 = target_item.astype(jnp.int32).reshape(B // 128, 128)
    histp = jnp.pad(hist_item.astype(jnp.int32), ((0, 0), (0, 256 - L)))

    def to_linear(t):
        mid = jax.lax.optimization_barrier(t.astype(jnp.float32).reshape(-1, 256))
        return mid.reshape(t.shape)

    urows, hsums, trows = _build_sc(B, L, nc, ns)(
        uid2, histp, tgt2, to_linear(user_table), to_linear(item_table))
    lenf = hist_len.astype(jnp.float32).reshape(B, 1)
    return _tc_mlp(urows, hsums, lenf, trows,
                   W1, b1.reshape(1, -1), W2, b2.reshape(1, -1))


# speedup vs baseline: 1.4856x; 1.4856x over previous
import functools

import jax
import jax.numpy as jnp
from jax import lax
from jax.experimental import pallas as pl
from jax.experimental.pallas import tpu as pltpu
from jax.experimental.pallas import tpu_sc as plsc

_D = 16
_CH = 8
_SLICES = ((0, 104), (104, 96))


def _build_sc_user(B, nc, ns):
    nw = nc * ns
    rw = B // nw
    grp = rw // 128

    mesh = plsc.VectorSubcoreMesh(core_axis_name="c", subcore_axis_name="s")

    @functools.partial(
        pl.kernel,
        out_type=jax.ShapeDtypeStruct((B, _D), jnp.float32),
        mesh=mesh,
        compiler_params=pltpu.CompilerParams(use_tc_tiling_on_sc=False),
        scratch_types=(
            pltpu.VMEM((grp, 128), jnp.int32),
            pltpu.VMEM((rw, _D), jnp.float32),
            pltpu.SemaphoreType.DMA,
        ),
    )
    def sc_user(uid_h, ut_h, uo_h, uidx, urows, gsem):
        wid = lax.axis_index("s") * nc + lax.axis_index("c")
        base = wid * rw
        pltpu.sync_copy(uid_h.at[pl.ds(wid * grp, grp)], uidx)
        handles = [pltpu.async_copy(
            ut_h.at[uidx.at[j]], urows.at[pl.ds(j * 128, 128)], gsem)
            for j in range(grp)]
        for h in handles:
            h.wait()
        pltpu.sync_copy(urows, uo_h.at[pl.ds(base, rw)])

    return sc_user


def _build_sc_hist(B, L, nc, ns):
    nw = nc * ns
    rw = B // nw
    nchunk = rw // _CH
    grp = rw // 128

    mesh = plsc.VectorSubcoreMesh(core_axis_name="c", subcore_axis_name="s")

    @functools.partial(
        pl.kernel,
        out_type=(
            jax.ShapeDtypeStruct((B, _D), jnp.float32),
            jax.ShapeDtypeStruct((B, _D), jnp.float32),
        ),
        mesh=mesh,
        compiler_params=pltpu.CompilerParams(use_tc_tiling_on_sc=False),
        scratch_types=(
            pltpu.VMEM((2, _CH, 256), jnp.int32),
            pltpu.VMEM((2, _CH, L, _D), jnp.float32),
            pltpu.VMEM((_CH, _D), jnp.float32),
            pltpu.VMEM((grp, 128), jnp.int32),
            pltpu.VMEM((rw, _D), jnp.float32),
            pltpu.SemaphoreType.DMA,
            pltpu.SemaphoreType.DMA,
        ),
    )
    def sc_fn(hist_h, tgt_h, it_h, ho_h, to_h,
              hidx, hrows, hsum, tidx, trows, hsem, gsem):
        wid = lax.axis_index("s") * nc + lax.axis_index("c")
        base = wid * rw

        pltpu.sync_copy(tgt_h.at[pl.ds(wid * grp, grp)], tidx)
        handles = [pltpu.async_copy(
            it_h.at[tidx.at[j]], trows.at[pl.ds(j * 128, 128)], gsem)
            for j in range(grp)]
        for h in handles:
            h.wait()
        pltpu.sync_copy(trows, to_h.at[pl.ds(base, rw)])

        zero = jnp.zeros((_D,), jnp.float32)

        def do_chunk(bb, chunk):
            row0 = base + chunk * _CH
            pltpu.sync_copy(hist_h.at[pl.ds(row0, _CH)], hidx.at[bb])
            hh = []
            for i in range(_CH):
                for (o, n) in _SLICES:
                    hh.append(pltpu.async_copy(
                        it_h.at[hidx.at[bb, i, pl.ds(o, n)]],
                        hrows.at[bb, i, pl.ds(o, n)],
                        hsem))
            for h in hh:
                h.wait()
            for i in range(_CH):
                def acc(j, a, i=i):
                    a0, a1, a2, a3 = a
                    b0 = j * 20
                    for k in range(0, 20, 4):
                        a0 = a0 + hrows[bb, i, b0 + k]
                        a1 = a1 + hrows[bb, i, b0 + k + 1]
                        a2 = a2 + hrows[bb, i, b0 + k + 2]
                        a3 = a3 + hrows[bb, i, b0 + k + 3]
                    return a0, a1, a2, a3
                a0, a1, a2, a3 = lax.fori_loop(
                    0, L // 20, acc, (zero, zero, zero, zero))
                hsum[i] = (a0 + a1) + (a2 + a3)
            pltpu.sync_copy(hsum, ho_h.at[pl.ds(row0, _CH)])

        def body(g, c):
            for bb in range(2):
                do_chunk(bb, g * 2 + bb)
            return c

        lax.fori_loop(0, nchunk // 2, body, 0)

    return sc_fn


def _tc_mlp(urows, hsums, lenf, trows, W1, b1, W2, b2):
    B = urows.shape[0]
    h1 = W1.shape[1]
    grid = 8
    rb = B // grid

    def body(u_ref, h_ref, l_ref, t_ref, w1_ref, b1_ref, w2_ref, b2_ref,
             uv_ref, iv_ref):
        denom = jnp.maximum(l_ref[...], 1.0)
        x = jnp.concatenate([u_ref[...], h_ref[...] / denom], axis=1)
        h = jnp.dot(x, w1_ref[...], preferred_element_type=jnp.float32)
        h = jnp.maximum(h + b1_ref[...], 0.0)
        h = jnp.dot(h, w2_ref[...], preferred_element_type=jnp.float32)
        h = jnp.maximum(h + b2_ref[...], 0.0)
        n = jnp.sqrt(jnp.sum(h * h, axis=1, keepdims=True))
        uv_ref[...] = h / jnp.maximum(n, 1e-12)
        t = t_ref[...]
        tn = jnp.sqrt(jnp.sum(t * t, axis=1, keepdims=True))
        iv_ref[...] = t / jnp.maximum(tn, 1e-12)

    out = pl.pallas_call(
        body,
        grid=(grid,),
        in_specs=[
            pl.BlockSpec((rb, _D), lambda i: (i, 0)),
            pl.BlockSpec((rb, _D), lambda i: (i, 0)),
            pl.BlockSpec((rb, 1), lambda i: (i, 0)),
            pl.BlockSpec((rb, _D), lambda i: (i, 0)),
            pl.BlockSpec((2 * _D, h1), lambda i: (0, 0)),
            pl.BlockSpec((1, h1), lambda i: (0, 0)),
            pl.BlockSpec((h1, _D), lambda i: (0, 0)),
            pl.BlockSpec((1, _D), lambda i: (0, 0)),
        ],
        out_specs=[
            pl.BlockSpec((rb, _D), lambda i: (i, 0)),
            pl.BlockSpec((rb, _D), lambda i: (i, 0)),
        ],
        out_shape=[
            jax.ShapeDtypeStruct((B, _D), jnp.float32),
            jax.ShapeDtypeStruct((B, _D), jnp.float32),
        ],
    )(urows, hsums, lenf, trows, W1, b1, W2, b2)
    return tuple(out)


def kernel(user_id, hist_item, hist_len, target_item, user_table, item_table,
           W1, b1, W2, b2):
    B, L = hist_item.shape
    info = plsc.get_sparse_core_info()
    nc, ns = info.num_cores, info.num_subcores
    uid2 = user_id.astype(jnp.int32).reshape(B // 128, 128)
    tgt2 = target_item.astype(jnp.int32).reshape(B // 128, 128)
    histp = jnp.pad(hist_item.astype(jnp.int32), ((0, 0), (0, 256 - L)))
    hsums, trows = _build_sc_hist(B, L, nc, ns)(
        histp, tgt2, item_table.astype(jnp.float32))
    urows = _build_sc_user(B, nc, ns)(uid2, user_table.astype(jnp.float32))
    lenf = hist_len.astype(jnp.float32).reshape(B, 1)
    return _tc_mlp(urows, hsums, lenf, trows,
                   W1, b1.reshape(1, -1), W2, b2.reshape(1, -1))
